# Initial kernel scaffold; baseline (speedup 1.0000x reference)
import jax
import jax.numpy as jnp
from jax.experimental import pallas as pl


def kernel(x, knn, W, bn_weight, bn_bias):
    raise NotImplementedError("write your pallas kernel here")


# speedup vs baseline: 22.9116x; 22.9116x over previous
import functools

import jax
import jax.numpy as jnp
from jax import lax
from jax.experimental import pallas as pl
from jax.experimental.pallas import tpu as pltpu
from jax.experimental.pallas import tpu_sc as plsc

B, N, K = 4, 10000, 16
D = 128
PTS = B * N
NC, NS = 2, 16
NW = NC * NS
CHUNK = 8
ROWS = CHUNK * K
NCHUNKS = PTS // CHUNK
NSLOT = (NCHUNKS + NW - 1) // NW + 1
NCOL = D // 16
BN_EPS = 1e-5


def _mm_body(x_ref, w_ref, h_ref):
    h_ref[...] = lax.dot_general(
        x_ref[...], w_ref[...],
        dimension_numbers=(((1,), (1,)), ((), ())),
        preferred_element_type=jnp.float32,
    )


def _matmul(x2d, w):
    br = 2000
    return pl.pallas_call(
        _mm_body,
        grid=(PTS // br,),
        in_specs=[
            pl.BlockSpec((br, D), lambda i: (i, 0)),
            pl.BlockSpec((D, D), lambda i: (0, 0)),
        ],
        out_specs=pl.BlockSpec((br, D), lambda i: (i, 0)),
        out_shape=jax.ShapeDtypeStruct((PTS, D), jnp.float32),
    )(x2d, w)


_MESH = plsc.VectorSubcoreMesh(
    core_axis_name="c", subcore_axis_name="s", num_cores=NC, num_subcores=NS)


@functools.partial(
    pl.kernel,
    out_type=(
        jax.ShapeDtypeStruct((PTS, D), jnp.float32),
        jax.ShapeDtypeStruct((NW, 2 * D), jnp.float32),
    ),
    mesh=_MESH,
    scratch_types=[
        pltpu.VMEM((2, ROWS), jnp.int32),
        pltpu.VMEM((2, ROWS, D), jnp.float32),
        pltpu.VMEM((2, CHUNK, D), jnp.float32),
        pltpu.VMEM((2 * D,), jnp.float32),
        pltpu.SemaphoreType.DMA,
        pltpu.SemaphoreType.DMA,
        pltpu.SemaphoreType.DMA,
        pltpu.SemaphoreType.DMA,
    ],
)
def _sc_gather_max(h_hbm, idx_hbm, m_hbm, part_hbm,
                   idx_v, rows_v, out_v, stat_v, gsem0, gsem1, osem0, osem1):
    cid = lax.axis_index("c")
    sid = lax.axis_index("s")
    wid = sid * NC + cid
    gsems = (gsem0, gsem1)
    osems = (osem0, osem1)

    def chunk_base(t):
        g = jnp.minimum(t * NW + wid, NCHUNKS - 1)
        return g * CHUNK

    def issue_gather(t, b):
        pltpu.sync_copy(idx_hbm.at[pl.ds(chunk_base(t) * K, ROWS)],
                        idx_v.at[b])
        pltpu.async_copy(h_hbm.at[idx_v.at[b]], rows_v.at[b], gsems[b])

    issue_gather(0, 0)
    issue_gather(1, 1)

    def chunk_compute(t, b, stats):
        validf = jnp.where(t * NW + wid < NCHUNKS, 1.0, 0.0).astype(jnp.float32)

        def point_body(p, carry):
            stats_in = carry
            acc = [rows_v[b, p * K, pl.ds(c * 16, 16)] for c in range(NCOL)]
            for j in range(1, K):
                for c in range(NCOL):
                    acc[c] = jnp.maximum(
                        acc[c], rows_v[b, p * K + j, pl.ds(c * 16, 16)])
            new_stats = []
            for c in range(NCOL):
                out_v[b, p, pl.ds(c * 16, 16)] = acc[c]
                masked = acc[c] * validf
                new_stats.append(stats_in[c] + masked)
                new_stats.append(stats_in[NCOL + c] + masked * acc[c])
            return tuple(new_stats[0::2]) + tuple(new_stats[1::2])

        return lax.fori_loop(0, CHUNK, point_body, stats)

    def outer_body(o, stats):
        for b in range(2):
            t = 2 * o + b
            pltpu.make_async_copy(
                h_hbm.at[idx_v.at[b]], rows_v.at[b], gsems[b]).wait()

            @pl.when(t >= 2)
            def _wait_out():
                pltpu.make_async_copy(
                    out_v.at[b],
                    m_hbm.at[pl.ds(chunk_base(t - 2), CHUNK)],
                    osems[b]).wait()

            stats = chunk_compute(t, b, stats)

            pltpu.async_copy(
                out_v.at[b], m_hbm.at[pl.ds(chunk_base(t), CHUNK)],
                osems[b])

            @pl.when(t + 2 < NSLOT)
            def _next_gather():
                issue_gather(t + 2, b)
        return stats

    zeros = tuple(jnp.zeros((16,), jnp.float32) for _ in range(2 * NCOL))
    stats = lax.fori_loop(0, NSLOT // 2, outer_body, zeros)

    for b in range(2):
        pltpu.make_async_copy(
            out_v.at[b],
            m_hbm.at[pl.ds(chunk_base(NSLOT - 2 + b), CHUNK)],
            osems[b]).wait()

    for c in range(NCOL):
        stat_v[pl.ds(c * 16, 16)] = stats[c]
        stat_v[pl.ds(D + c * 16, 16)] = stats[NCOL + c]
    pltpu.sync_copy(stat_v, part_hbm.at[wid])


def _bn_body(m_ref, part_ref, bnw_ref, bnb_ref, y_ref):
    part = part_ref[...]
    total = jnp.sum(part, axis=0, keepdims=True)
    mean = total[:, :D] / PTS
    var = total[:, D:] / PTS - mean * mean
    scale = bnw_ref[...] * lax.rsqrt(var + BN_EPS)
    off = bnb_ref[...] - mean * scale
    y_ref[...] = m_ref[...] * scale + off


def _batchnorm(m, part, bnw, bnb):
    br = 2000
    return pl.pallas_call(
        _bn_body,
        grid=(PTS // br,),
        in_specs=[
            pl.BlockSpec((br, D), lambda i: (i, 0)),
            pl.BlockSpec((NW, 2 * D), lambda i: (0, 0)),
            pl.BlockSpec((1, D), lambda i: (0, 0)),
            pl.BlockSpec((1, D), lambda i: (0, 0)),
        ],
        out_specs=pl.BlockSpec((br, D), lambda i: (i, 0)),
        out_shape=jax.ShapeDtypeStruct((PTS, D), jnp.float32),
    )(m, part, bnw, bnb)


def kernel(x, knn, W, bn_weight, bn_bias):
    x2d = x.reshape(PTS, D)
    h = _matmul(x2d, W)
    glob = (knn + (jnp.arange(B, dtype=jnp.int32) * N)[:, None, None])
    idx_flat = glob.reshape(PTS * K)
    m, part = _sc_gather_max(h, idx_flat)
    y = _batchnorm(m, part, bn_weight.reshape(1, D), bn_bias.reshape(1, D))
    return y.reshape(B, N, D)


# speedup vs baseline: 38.1696x; 1.6660x over previous
import functools

import jax
import jax.numpy as jnp
from jax import lax
from jax.experimental import pallas as pl
from jax.experimental.pallas import tpu as pltpu
from jax.experimental.pallas import tpu_sc as plsc

B, N, K = 4, 10000, 16
D = 128
PTS = B * N
NC, NS = 2, 16
NW = NC * NS
CHUNK = 8
ROWS = CHUNK * K
NCHUNKS = PTS // CHUNK
NBUF = 4
NSLOT = -(-((NCHUNKS + NW - 1) // NW) // NBUF) * NBUF
NCOL = D // 16
BN_EPS = 1e-5


def _mm_body(x_ref, w_ref, h_ref):
    h_ref[...] = lax.dot_general(
        x_ref[...], w_ref[...],
        dimension_numbers=(((1,), (1,)), ((), ())),
        preferred_element_type=jnp.float32,
    )


def _matmul(x2d, w):
    br = 2000
    return pl.pallas_call(
        _mm_body,
        grid=(PTS // br,),
        in_specs=[
            pl.BlockSpec((br, D), lambda i: (i, 0)),
            pl.BlockSpec((D, D), lambda i: (0, 0)),
        ],
        out_specs=pl.BlockSpec((br, D), lambda i: (i, 0)),
        out_shape=jax.ShapeDtypeStruct((PTS, D), jnp.float32),
    )(x2d, w)


_MESH = plsc.VectorSubcoreMesh(
    core_axis_name="c", subcore_axis_name="s", num_cores=NC, num_subcores=NS)


@functools.partial(
    pl.kernel,
    out_type=(
        jax.ShapeDtypeStruct((PTS, D), jnp.float32),
        jax.ShapeDtypeStruct((NW, 2 * D), jnp.float32),
    ),
    mesh=_MESH,
    scratch_types=[
        pltpu.VMEM((NBUF, ROWS), jnp.int32),
        pltpu.VMEM((NBUF, ROWS, D), jnp.float32),
        pltpu.VMEM((NBUF, CHUNK, D), jnp.float32),
        pltpu.VMEM((2 * D,), jnp.float32),
        [pltpu.SemaphoreType.DMA] * NBUF,
        [pltpu.SemaphoreType.DMA] * NBUF,
        [pltpu.SemaphoreType.DMA] * NBUF,
    ],
)
def _sc_gather_max(h_hbm, idx_hbm, m_hbm, part_hbm,
                   idx_v, rows_v, out_v, stat_v, isems, gsems, osems):
    cid = lax.axis_index("c")
    sid = lax.axis_index("s")
    wid = sid * NC + cid

    def chunk_base(t):
        g = jnp.minimum(t * NW + wid, NCHUNKS - 1)
        return g * CHUNK

    def issue_idx(t, b):
        pltpu.async_copy(idx_hbm.at[pl.ds(chunk_base(t) * K, ROWS)],
                         idx_v.at[b], isems[b])

    def issue_gather(t, b):
        pltpu.make_async_copy(idx_hbm.at[pl.ds(chunk_base(t) * K, ROWS)],
                              idx_v.at[b], isems[b]).wait()
        pltpu.async_copy(h_hbm.at[idx_v.at[b]], rows_v.at[b], gsems[b])

    for b in range(NBUF):
        issue_idx(b, b)
    for b in range(NBUF - 1):
        issue_gather(b, b)

    def chunk_compute(t, b, stats):
        validf = jnp.where(t * NW + wid < NCHUNKS, 1.0, 0.0).astype(jnp.float32)

        def point_body(p, carry):
            stats_in = carry
            acc = [rows_v[b, p * K, pl.ds(c * 16, 16)] for c in range(NCOL)]
            for j in range(1, K):
                for c in range(NCOL):
                    acc[c] = jnp.maximum(
                        acc[c], rows_v[b, p * K + j, pl.ds(c * 16, 16)])
            new_stats = []
            for c in range(NCOL):
                out_v[b, p, pl.ds(c * 16, 16)] = acc[c]
                masked = acc[c] * validf
                new_stats.append(stats_in[c] + masked)
                new_stats.append(stats_in[NCOL + c] + masked * acc[c])
            return tuple(new_stats[0::2]) + tuple(new_stats[1::2])

        return lax.fori_loop(0, CHUNK, point_body, stats)

    def outer_body(o, stats):
        for b in range(NBUF):
            t = o * NBUF + b
            bp = (b - 1) % NBUF

            @pl.when(t + NBUF - 1 < NSLOT)
            def _front_gather():
                issue_gather(t + NBUF - 1, bp)

            pltpu.make_async_copy(
                h_hbm.at[idx_v.at[b]], rows_v.at[b], gsems[b]).wait()

            @pl.when(t + NBUF < NSLOT)
            def _front_idx():
                issue_idx(t + NBUF, b)

            @pl.when(t >= NBUF)
            def _wait_out():
                pltpu.make_async_copy(
                    out_v.at[b],
                    m_hbm.at[pl.ds(chunk_base(t - NBUF), CHUNK)],
                    osems[b]).wait()

            stats = chunk_compute(t, b, stats)

            pltpu.async_copy(
                out_v.at[b], m_hbm.at[pl.ds(chunk_base(t), CHUNK)],
                osems[b])
        return stats

    zeros = tuple(jnp.zeros((16,), jnp.float32) for _ in range(2 * NCOL))
    stats = lax.fori_loop(0, NSLOT // NBUF, outer_body, zeros)

    for b in range(NBUF):
        pltpu.make_async_copy(
            out_v.at[b],
            m_hbm.at[pl.ds(chunk_base(NSLOT - NBUF + b), CHUNK)],
            osems[b]).wait()

    for c in range(NCOL):
        stat_v[pl.ds(c * 16, 16)] = stats[c]
        stat_v[pl.ds(D + c * 16, 16)] = stats[NCOL + c]
    pltpu.sync_copy(stat_v, part_hbm.at[wid])


def _bn_body(m_ref, part_ref, bnw_ref, bnb_ref, y_ref):
    part = part_ref[...]
    total = jnp.sum(part, axis=0, keepdims=True)
    mean = total[:, :D] / PTS
    var = total[:, D:] / PTS - mean * mean
    scale = bnw_ref[...] * lax.rsqrt(var + BN_EPS)
    off = bnb_ref[...] - mean * scale
    y_ref[...] = m_ref[...] * scale + off


def _batchnorm(m, part, bnw, bnb):
    br = 2000
    return pl.pallas_call(
        _bn_body,
        grid=(PTS // br,),
        in_specs=[
            pl.BlockSpec((br, D), lambda i: (i, 0)),
            pl.BlockSpec((NW, 2 * D), lambda i: (0, 0)),
            pl.BlockSpec((1, D), lambda i: (0, 0)),
            pl.BlockSpec((1, D), lambda i: (0, 0)),
        ],
        out_specs=pl.BlockSpec((br, D), lambda i: (i, 0)),
        out_shape=jax.ShapeDtypeStruct((PTS, D), jnp.float32),
    )(m, part, bnw, bnb)


def kernel(x, knn, W, bn_weight, bn_bias):
    x2d = x.reshape(PTS, D)
    h = _matmul(x2d, W)
    glob = (knn + (jnp.arange(B, dtype=jnp.int32) * N)[:, None, None])
    idx_flat = glob.reshape(PTS * K)
    m, part = _sc_gather_max(h, idx_flat)
    y = _batchnorm(m, part, bn_weight.reshape(1, D), bn_bias.reshape(1, D))
    return y.reshape(B, N, D)
